# baseline (speedup 1.0000x reference)
import functools

import jax
import jax.numpy as jnp
from jax import lax
from jax.experimental import pallas as pl
from jax.experimental.pallas import tpu as pltpu
from jax.experimental.pallas import tpu_sc as plsc

_N_FIELDS = 26
_VOCAB = 100000
_EMB = 64
_NUM_DIM = 13
_BATCH = 16384
_TOK = _N_FIELDS + 1

_NC = 2
_NS = 16
_NW = _NC * _NS

_LANES = 16
_SUB = 8

_ROWS = _BATCH * _N_FIELDS
_NB_W = _BATCH // _NW
_QB = 1
_QROWS = _QB * _N_FIELDS
_OB = 4
_GPB = _OB // _QB
_NBLK = _NB_W // _OB
_NGRP = _NB_W // _QB
_XPW = _NB_W * _N_FIELDS


def _num_matmul(x_num, W, b):
    bm = 2048

    def body(x_ref, w_ref, b_ref, o_ref):
        acc = (
            jnp.dot(x_ref[...], w_ref[...], preferred_element_type=jnp.float32)
            + b_ref[...]
        )
        o_ref[:, 0:_EMB] = acc

    return pl.pallas_call(
        body,
        out_shape=jax.ShapeDtypeStruct((_BATCH, 2 * _EMB), jnp.float32),
        grid=(_BATCH // bm,),
        in_specs=[
            pl.BlockSpec((bm, _NUM_DIM), lambda i: (i, 0)),
            pl.BlockSpec((_NUM_DIM, _EMB), lambda i: (0, 0)),
            pl.BlockSpec((1, _EMB), lambda i: (0, 0)),
        ],
        out_specs=pl.BlockSpec((bm, 2 * _EMB), lambda i: (i, 0)),
    )(x_num, W, b.reshape(1, _EMB))


def _sc_body(xcat_hbm, num_hbm, table_hbm, out_hbm,
             xbuf, gbuf8, cbuf, numbuf,
             gsem0, gsem1, osem0, osem1, nsem):
    wid = lax.axis_index("s") * _NC + lax.axis_index("c")

    pltpu.sync_copy(xcat_hbm.at[pl.ds(wid * _XPW, _XPW)],
                    xbuf.at[pl.ds(0, _XPW)])

    gsems = (gsem0, gsem1)
    osems = (osem0, osem1)

    def start_group(g, slot):
        base = g * _QROWS

        def field(f, carry):
            v = xbuf[pl.ds(base + f, _LANES)][0]
            v8 = pl.multiple_of((v // _SUB) * _SUB, _SUB)
            pltpu.async_copy(
                table_hbm.at[f, pl.ds(v8, _SUB), :],
                gbuf8.at[slot, f],
                gsems[slot],
            )
            return carry

        lax.fori_loop(0, _N_FIELDS, field, 0)

    def wait_group(slot):
        pltpu.make_async_copy(
            table_hbm.at[0, pl.ds(0, _SUB * _QROWS), :].reshape(
                _QROWS, _SUB, _EMB),
            gbuf8.at[slot],
            gsems[slot],
        ).wait()

    def compact(g, slot, ib, q):
        base = g * _QROWS

        def field(f, carry):
            voff = lax.rem(xbuf[pl.ds(base + f, _LANES)][0], _SUB)
            for eg in range(_EMB // _LANES):
                cbuf[ib, q, f, pl.ds(eg * _LANES, _LANES)] = (
                    gbuf8[slot, f, voff, pl.ds(eg * _LANES, _LANES)]
                )
            return carry

        lax.fori_loop(0, _N_FIELDS, field, 0)

    def do_block(i, ib):
        blk = 2 * i + ib
        b0 = wid * _NB_W + blk * _OB

        @pl.when(i > 0)
        def _():
            pltpu.make_async_copy(
                cbuf.at[ib], out_hbm.at[pl.ds(b0 - 2 * _OB, _OB)], osems[ib]
            ).wait()

        pltpu.async_copy(
            num_hbm.at[pl.ds(b0, _OB), :], numbuf.at[ib], nsem)

        for q in range(_GPB):
            g = blk * _GPB + q
            slot = q % 2

            @pl.when(g + 1 < _NGRP)
            def _():
                start_group(g + 1, (q + 1) % 2)

            wait_group(slot)
            compact(g, slot, ib, q)

        pltpu.make_async_copy(
            num_hbm.at[pl.ds(b0, _OB), :], numbuf.at[ib], nsem).wait()
        for bl in range(_OB):
            for eg in range(_EMB // _LANES):
                cbuf[ib, bl, _N_FIELDS, pl.ds(eg * _LANES, _LANES)] = (
                    numbuf[ib, bl, pl.ds(eg * _LANES, _LANES)]
                )
        pltpu.async_copy(
            cbuf.at[ib], out_hbm.at[pl.ds(b0, _OB)], osems[ib])

    start_group(0, 0)

    def pair(i, carry):
        do_block(i, 0)
        do_block(i, 1)
        return carry

    lax.fori_loop(0, _NBLK // 2, pair, 0)

    for ib in range(2):
        b0 = wid * _NB_W + (_NBLK - 2 + ib) * _OB
        pltpu.make_async_copy(
            cbuf.at[ib], out_hbm.at[pl.ds(b0, _OB)], osems[ib]
        ).wait()


_sc_gather = functools.partial(
    pl.kernel,
    out_type=jax.ShapeDtypeStruct((_BATCH, _TOK, _EMB), jnp.float32),
    mesh=plsc.VectorSubcoreMesh(core_axis_name="c", subcore_axis_name="s"),
    scratch_types=[
        pltpu.VMEM((_XPW + _LANES,), jnp.int32),
        pltpu.VMEM((2, _QROWS, _SUB, _EMB), jnp.float32),
        pltpu.VMEM((2, _OB, _TOK, _EMB), jnp.float32),
        pltpu.VMEM((2, _OB, 2 * _EMB), jnp.float32),
        pltpu.SemaphoreType.DMA,
        pltpu.SemaphoreType.DMA,
        pltpu.SemaphoreType.DMA,
        pltpu.SemaphoreType.DMA,
        pltpu.SemaphoreType.DMA,
    ],
)(_sc_body)


def kernel(x_cat, x_num, tables, W, b):
    num = _num_matmul(x_num, W, b)
    xflat = x_cat.astype(jnp.int32).reshape(_ROWS)
    return _sc_gather(xflat, num, tables)


# speedup vs baseline: 1.0118x; 1.0118x over previous
import functools

import jax
import jax.numpy as jnp
from jax import lax
from jax.experimental import pallas as pl
from jax.experimental.pallas import tpu as pltpu
from jax.experimental.pallas import tpu_sc as plsc

_N_FIELDS = 26
_VOCAB = 100000
_EMB = 64
_NUM_DIM = 13
_BATCH = 16384
_TOK = _N_FIELDS + 1

_NC = 2
_NS = 16
_NW = _NC * _NS

_LANES = 16
_SUB = 8

_ROWS = _BATCH * _N_FIELDS
_NB_W = _BATCH // _NW
_QB = 1
_QROWS = _QB * _N_FIELDS
_OB = 4
_GPB = _OB // _QB
_NBLK = _NB_W // _OB
_NGRP = _NB_W // _QB
_XPW = _NB_W * _N_FIELDS


def _num_matmul(x_num, W, b):
    bm = 2048

    def body(x_ref, w_ref, b_ref, o_ref):
        acc = (
            jnp.dot(x_ref[...], w_ref[...], preferred_element_type=jnp.float32)
            + b_ref[...]
        )
        o_ref[:, 0:_EMB] = acc

    return pl.pallas_call(
        body,
        out_shape=jax.ShapeDtypeStruct((_BATCH, 2 * _EMB), jnp.float32),
        grid=(_BATCH // bm,),
        in_specs=[
            pl.BlockSpec((bm, _NUM_DIM), lambda i: (i, 0)),
            pl.BlockSpec((_NUM_DIM, _EMB), lambda i: (0, 0)),
            pl.BlockSpec((1, _EMB), lambda i: (0, 0)),
        ],
        out_specs=pl.BlockSpec((bm, 2 * _EMB), lambda i: (i, 0)),
    )(x_num, W, b.reshape(1, _EMB))


def _sc_body(xcat_hbm, num_hbm, table_hbm, out_hbm,
             xbuf, gbuf8, cbuf, numbuf,
             gsem0, gsem1, osem0, osem1, nsem):
    wid = lax.axis_index("s") * _NC + lax.axis_index("c")

    pltpu.sync_copy(xcat_hbm.at[pl.ds(wid * _XPW, _XPW)],
                    xbuf.at[pl.ds(0, _XPW)])

    gsems = (gsem0, gsem1)
    osems = (osem0, osem1)

    def start_group(g, slot):
        base = g * _QROWS

        def field(f, carry):
            v = xbuf[pl.ds(base + f, _LANES)][0]
            v8 = pl.multiple_of(lax.bitwise_and(v, -_SUB), _SUB)
            pltpu.async_copy(
                table_hbm.at[f, pl.ds(v8, _SUB), :],
                gbuf8.at[slot, f],
                gsems[slot],
            )
            return carry

        lax.fori_loop(0, _N_FIELDS, field, 0, unroll=2)

    def wait_group(slot):
        pltpu.make_async_copy(
            table_hbm.at[0, pl.ds(0, _SUB * _QROWS), :].reshape(
                _QROWS, _SUB, _EMB),
            gbuf8.at[slot],
            gsems[slot],
        ).wait()

    def compact(g, slot, ib, q):
        base = g * _QROWS

        def field(f, carry):
            voff = lax.bitwise_and(xbuf[pl.ds(base + f, _LANES)][0], _SUB - 1)
            for eg in range(_EMB // _LANES):
                cbuf[ib, q, f, pl.ds(eg * _LANES, _LANES)] = (
                    gbuf8[slot, f, voff, pl.ds(eg * _LANES, _LANES)]
                )
            return carry

        lax.fori_loop(0, _N_FIELDS, field, 0, unroll=2)

    def do_block(i, ib):
        blk = 2 * i + ib
        b0 = wid * _NB_W + blk * _OB

        @pl.when(i > 0)
        def _():
            pltpu.make_async_copy(
                cbuf.at[ib], out_hbm.at[pl.ds(b0 - 2 * _OB, _OB)], osems[ib]
            ).wait()

        pltpu.async_copy(
            num_hbm.at[pl.ds(b0, _OB), :], numbuf.at[ib], nsem)

        for q in range(_GPB):
            g = blk * _GPB + q
            slot = q % 2

            @pl.when(g + 1 < _NGRP)
            def _():
                start_group(g + 1, (q + 1) % 2)

            wait_group(slot)
            compact(g, slot, ib, q)

        pltpu.make_async_copy(
            num_hbm.at[pl.ds(b0, _OB), :], numbuf.at[ib], nsem).wait()
        for bl in range(_OB):
            for eg in range(_EMB // _LANES):
                cbuf[ib, bl, _N_FIELDS, pl.ds(eg * _LANES, _LANES)] = (
                    numbuf[ib, bl, pl.ds(eg * _LANES, _LANES)]
                )
        pltpu.async_copy(
            cbuf.at[ib], out_hbm.at[pl.ds(b0, _OB)], osems[ib])

    start_group(0, 0)

    def pair(i, carry):
        do_block(i, 0)
        do_block(i, 1)
        return carry

    lax.fori_loop(0, _NBLK // 2, pair, 0)

    for ib in range(2):
        b0 = wid * _NB_W + (_NBLK - 2 + ib) * _OB
        pltpu.make_async_copy(
            cbuf.at[ib], out_hbm.at[pl.ds(b0, _OB)], osems[ib]
        ).wait()


_sc_gather = functools.partial(
    pl.kernel,
    out_type=jax.ShapeDtypeStruct((_BATCH, _TOK, _EMB), jnp.float32),
    mesh=plsc.VectorSubcoreMesh(core_axis_name="c", subcore_axis_name="s"),
    scratch_types=[
        pltpu.VMEM((_XPW + _LANES,), jnp.int32),
        pltpu.VMEM((2, _QROWS, _SUB, _EMB), jnp.float32),
        pltpu.VMEM((2, _OB, _TOK, _EMB), jnp.float32),
        pltpu.VMEM((2, _OB, 2 * _EMB), jnp.float32),
        pltpu.SemaphoreType.DMA,
        pltpu.SemaphoreType.DMA,
        pltpu.SemaphoreType.DMA,
        pltpu.SemaphoreType.DMA,
        pltpu.SemaphoreType.DMA,
    ],
)(_sc_body)


def kernel(x_cat, x_num, tables, W, b):
    num = _num_matmul(x_num, W, b)
    xflat = x_cat.astype(jnp.int32).reshape(_ROWS)
    return _sc_gather(xflat, num, tables)
